# baseline (speedup 1.0000x reference)
import functools

import jax
import jax.numpy as jnp
from jax import lax
from jax.experimental import pallas as pl
from jax.experimental.pallas import tpu as pltpu
from jax.experimental.pallas import tpu_sc as plsc

B = 16384
D = 32
K = 64
ALPHA = 0.1
REG_W = 1e-5

_NC = 2
_NS = 16
_NW = _NC * _NS
_BPW = B // _NW
_CH = 128
_NCH = _BPW // _CH


def _gather3_sc(uid2, pos2, neg2, utab, itab):
    mesh = plsc.VectorSubcoreMesh(core_axis_name="c", subcore_axis_name="s")

    @functools.partial(
        pl.kernel,
        mesh=mesh,
        compiler_params=pltpu.CompilerParams(use_tc_tiling_on_sc=False),
        out_type=[jax.ShapeDtypeStruct((B, D), jnp.float32)] * 3,
        scratch_types=[
            pltpu.VMEM((_NCH, _CH), jnp.int32),
            pltpu.VMEM((_NCH, _CH), jnp.int32),
            pltpu.VMEM((_NCH, _CH), jnp.int32),
            pltpu.VMEM((_BPW, D), jnp.float32),
            pltpu.VMEM((_BPW, D), jnp.float32),
            pltpu.VMEM((_BPW, D), jnp.float32),
            pltpu.SemaphoreType.DMA,
        ],
    )
    def k(uid_h, pos_h, neg_h, utab_h, itab_h, u_out, p_out, n_out,
          uidx, pidx, nidx, urows, prows, nrows, sem):
        wid = lax.axis_index("s") * _NC + lax.axis_index("c")
        base = wid * _BPW
        pltpu.sync_copy(uid_h.at[wid], uidx)
        pltpu.sync_copy(pos_h.at[wid], pidx)
        pltpu.sync_copy(neg_h.at[wid], nidx)
        copies = []
        for j in range(_NCH):
            sl = pl.ds(j * _CH, _CH)
            copies.append(pltpu.async_copy(utab_h.at[uidx.at[j]], urows.at[sl], sem))
            copies.append(pltpu.async_copy(itab_h.at[pidx.at[j]], prows.at[sl], sem))
            copies.append(pltpu.async_copy(itab_h.at[nidx.at[j]], nrows.at[sl], sem))
        for c in copies:
            c.wait()
        pltpu.sync_copy(urows, u_out.at[pl.ds(base, _BPW)])
        pltpu.sync_copy(prows, p_out.at[pl.ds(base, _BPW)])
        pltpu.sync_copy(nrows, n_out.at[pl.ds(base, _BPW)])

    return k(uid2, pos2, neg2, utab, itab)


def _loss_tc_body(u_ref, p_ref, n_ref, c_ref, o_ref):
    u = u_ref[...]
    p = p_ref[...]
    n = n_ref[...]
    c = c_ref[...]
    delta = jnp.sum(u * (p - n), axis=1)
    sig = jnp.where(delta >= 0.0,
                    1.0 / (1.0 + jnp.exp(-delta)),
                    jnp.exp(delta) / (1.0 + jnp.exp(delta)))
    bpr_sum = jnp.sum(jnp.log(sig + 1e-8))
    uu = jnp.sum(u * u, axis=1)
    reg_sum = jnp.sum(uu) + jnp.sum(p * p) + jnp.sum(n * n)
    w = 1.0 / jnp.maximum(jnp.sqrt(uu), 1e-12)
    sumfeat = jnp.sum(u * w[:, None], axis=0)
    sumfeatsq = jnp.sum(uu * w * w)
    cw = 1.0 / jnp.maximum(jnp.sqrt(jnp.sum(c * c, axis=1)), 1e-12)
    cb = c * cw[:, None]
    s_cc = jnp.sum(cb * cb)
    sum_cb = jnp.sum(cb, axis=0)
    sdl = sumfeatsq / B + s_cc / K - (2.0 / (B * K)) * jnp.sum(sumfeat * sum_cb)
    com = (2.0 * K * s_cc - 2.0 * jnp.sum(sum_cb * sum_cb)) / (K * (K - 1))
    bpr = -bpr_sum / B
    total = bpr + ALPHA * (sdl - com) + REG_W * reg_sum
    o_ref[...] = jnp.reshape(total, (1, 1))


def _loss_tc(u, p, n, c):
    return pl.pallas_call(
        _loss_tc_body,
        out_shape=jax.ShapeDtypeStruct((1, 1), jnp.float32),
    )(u, p, n, c)


def kernel(uid, src_ids, pos_ids, neg_ids, src_user_emb, src_item_emb, src_clusters):
    del src_ids
    uid2 = uid.astype(jnp.int32).reshape(_NW, _NCH, _CH)
    pos2 = pos_ids.astype(jnp.int32).reshape(_NW, _NCH, _CH)
    neg2 = neg_ids.astype(jnp.int32).reshape(_NW, _NCH, _CH)
    u, p, n = _gather3_sc(uid2, pos2, neg2, src_user_emb, src_item_emb)
    out = _loss_tc(u, p, n, src_clusters)
    return out[0, 0]


# speedup vs baseline: 9.3093x; 9.3093x over previous
import jax
import jax.numpy as jnp
from jax.experimental import pallas as pl

B = 16384
D = 32
K = 64
ALPHA = 0.1
REG_W = 1e-5


def _loss_body(ut_ref, pt_ref, nt_ref, ct_ref, o_ref):
    ut = ut_ref[...]
    pt = pt_ref[...]
    nt = nt_ref[...]
    ct = ct_ref[...]
    delta = jnp.sum(ut * (pt - nt), axis=0)
    sig = jnp.where(delta >= 0.0,
                    1.0 / (1.0 + jnp.exp(-delta)),
                    jnp.exp(delta) / (1.0 + jnp.exp(delta)))
    bpr_sum = jnp.sum(jnp.log(sig + 1e-8))
    uu = jnp.sum(ut * ut, axis=0)
    reg_sum = jnp.sum(uu) + jnp.sum(pt * pt) + jnp.sum(nt * nt)
    w = 1.0 / jnp.maximum(jnp.sqrt(uu), 1e-12)
    sumfeat = jnp.sum(ut * w[None, :], axis=1)
    sumfeatsq = jnp.sum(uu * w * w)
    cw = 1.0 / jnp.maximum(jnp.sqrt(jnp.sum(ct * ct, axis=0)), 1e-12)
    cb = ct * cw[None, :]
    s_cc = jnp.sum(cb * cb)
    sum_cb = jnp.sum(cb, axis=1)
    sdl = sumfeatsq / B + s_cc / K - (2.0 / (B * K)) * jnp.sum(sumfeat * sum_cb)
    com = (2.0 * K * s_cc - 2.0 * jnp.sum(sum_cb * sum_cb)) / (K * (K - 1))
    total = -bpr_sum / B + ALPHA * (sdl - com) + REG_W * reg_sum
    o_ref[...] = jnp.reshape(total, (1, 1))


def kernel(uid, src_ids, pos_ids, neg_ids, src_user_emb, src_item_emb, src_clusters):
    del src_ids
    u = jnp.take(src_user_emb, uid, axis=0)
    p = jnp.take(src_item_emb, pos_ids, axis=0)
    n = jnp.take(src_item_emb, neg_ids, axis=0)
    out = pl.pallas_call(
        _loss_body,
        out_shape=jax.ShapeDtypeStruct((1, 1), jnp.float32),
    )(u.T, p.T, n.T, src_clusters.T)
    return out[0, 0]


# speedup vs baseline: 9.7458x; 1.0469x over previous
import jax
import jax.numpy as jnp
from jax.experimental import pallas as pl

B = 16384
D = 32
K = 64
ALPHA = 0.1
REG_W = 1e-5


def _loss_body(ut_ref, pt_ref, nt_ref, ct_ref, o_ref):
    ut = ut_ref[...]
    pt = pt_ref[...]
    nt = nt_ref[...]
    ct = ct_ref[...]
    delta = jnp.sum(ut * (pt - nt), axis=0)
    sig = jnp.where(delta >= 0.0,
                    1.0 / (1.0 + jnp.exp(-delta)),
                    jnp.exp(delta) / (1.0 + jnp.exp(delta)))
    bpr_sum = jnp.sum(jnp.log(sig + 1e-8))
    uu = jnp.sum(ut * ut, axis=0)
    reg_sum = jnp.sum(uu) + jnp.sum(pt * pt) + jnp.sum(nt * nt)
    w = 1.0 / jnp.maximum(jnp.sqrt(uu), 1e-12)
    sumfeat = jnp.sum(ut * w[None, :], axis=1)
    sumfeatsq = jnp.sum(uu * w * w)
    cw = 1.0 / jnp.maximum(jnp.sqrt(jnp.sum(ct * ct, axis=0)), 1e-12)
    cb = ct * cw[None, :]
    s_cc = jnp.sum(cb * cb)
    sum_cb = jnp.sum(cb, axis=1)
    sdl = sumfeatsq / B + s_cc / K - (2.0 / (B * K)) * jnp.sum(sumfeat * sum_cb)
    com = (2.0 * K * s_cc - 2.0 * jnp.sum(sum_cb * sum_cb)) / (K * (K - 1))
    total = -bpr_sum / B + ALPHA * (sdl - com) + REG_W * reg_sum
    o_ref[...] = jnp.reshape(total, (1, 1))


def _rows(table, idx):
    dnums = jax.lax.GatherDimensionNumbers(
        offset_dims=(1,), collapsed_slice_dims=(0,), start_index_map=(0,))
    return jax.lax.gather(
        table, idx[:, None], dnums, slice_sizes=(1, table.shape[1]),
        mode=jax.lax.GatherScatterMode.PROMISE_IN_BOUNDS)


def kernel(uid, src_ids, pos_ids, neg_ids, src_user_emb, src_item_emb, src_clusters):
    del src_ids
    u = _rows(src_user_emb, uid)
    p = _rows(src_item_emb, pos_ids)
    n = _rows(src_item_emb, neg_ids)
    out = pl.pallas_call(
        _loss_body,
        out_shape=jax.ShapeDtypeStruct((1, 1), jnp.float32),
    )(u.T, p.T, n.T, src_clusters.T)
    return out[0, 0]
